# baseline (speedup 1.0000x reference)
import jax
import jax.numpy as jnp
from jax import lax
from jax.experimental import pallas as pl
from jax.experimental.pallas import tpu as pltpu
from jax.experimental.pallas import tpu_sc as plsc

N0 = 5000
N = 10000
E = 320000
D = 128
DH = 64
NUM_ETYPES = 4
HEADS = 2

NC = 2
NS = 16
L = 16

NP = 10240
RPT = NP // NS

E2 = 327680
EPT = E2 // NS
CH = 128
NR2 = EPT // CH
GROWS = 2
NGRP = NR2 // GROWS
SBLK = 64

DW = 1024
DR = NP // DW
NV = HEADS * NUM_ETYPES


_BLK_A = 1280


def _fc1_body(feat_ref, w0_ref, b0_ref, w1_ref, b1_ref, sc_ref, out_ref):
    i = pl.program_id(0)
    feat = feat_ref[...]
    h0 = jnp.dot(feat, w0_ref[...].T, preferred_element_type=jnp.float32) + b0_ref[...]
    h1 = jnp.dot(feat, w1_ref[...].T, preferred_element_type=jnp.float32) + b1_ref[...]
    rows = jax.lax.broadcasted_iota(jnp.int32, (_BLK_A, 1), 0) + i * _BLK_A
    h = jnp.where(rows < N0, h0, h1)
    for hf in range(2):
        hh = h[:, hf * DH:(hf + 1) * DH]
        for v in range(NV):
            out_ref[hf, v, :, :] = hh * sc_ref[v:v + 1, :DH]


def _fc1_scaled_tables(featp, w0, b0, w1, b1, scales):
    nblk = NP // _BLK_A
    return pl.pallas_call(
        _fc1_body,
        grid=(nblk,),
        in_specs=[
            pl.BlockSpec((_BLK_A, D), lambda i: (i, 0)),
            pl.BlockSpec((D, D), lambda i: (0, 0)),
            pl.BlockSpec((1, D), lambda i: (0, 0)),
            pl.BlockSpec((D, D), lambda i: (0, 0)),
            pl.BlockSpec((1, D), lambda i: (0, 0)),
            pl.BlockSpec((NV, D), lambda i: (0, 0)),
        ],
        out_specs=pl.BlockSpec((2, NV, _BLK_A, DH), lambda i: (0, 0, i, 0)),
        out_shape=jax.ShapeDtypeStruct((2, NV, NP, DH), jnp.float32),
    )(featp, w0, b0, w1, b1, scales)


def _zero_2d(ref, nrows, ncols):
    z = jnp.zeros((L,), jnp.float32)
    for r in range(nrows):
        @pl.loop(0, ncols // L)
        def _(j):
            ref[r, pl.ds(j * L, L)] = z


def _sc_body(t1a, t1b, epkv, iotav, filtv, zz, t2a, t2b,
             epk, idx_a, idx_b, sc0, sc1, sc2, sc3, rows_a, rows_b, sbuf,
             degl, degf, floc1, floc2, iotab, idx10, recrow,
             acc, dacc, stsem, gsem, ssem):
    idxg = [idx_a, idx_b]
    scx = [sc0, sc1, sc2, sc3]
    rows = [rows_a, rows_b]
    cid = lax.axis_index("c")
    wid = lax.axis_index("s")
    base = wid * RPT

    pltpu.sync_copy(filtv.at[cid, 0], floc1)
    pltpu.sync_copy(filtv.at[cid, 1], floc2)
    iotab[...] = lax.iota(jnp.int32, L)
    pltpu.sync_copy(iotav.at[pl.ds(0, DR)], idx10)

    voff1 = (cid * NUM_ETYPES) * NP

    def zero_acc_slice():
        _zero_2d(sbuf, SBLK, DH)
        for k in range(RPT // SBLK):
            pltpu.sync_copy(sbuf, acc.at[pl.ds(base + k * SBLK, SBLK), :])

    def _stage(g, p):
        rb = wid * NR2 + g * GROWS
        pltpu.async_copy(epkv.at[pl.ds(rb, GROWS)], epk.at[p], stsem)

    def _wait_stage(p):
        sl = pl.ds(wid * NR2, GROWS)
        pltpu.make_async_copy(epkv.at[sl], epk.at[p], stsem).wait()

    def _fire_gather(table, p):
        pltpu.async_copy(table.at[idxg[p]], rows[p], gsem)

    def _wait_gather(table, p):
        pltpu.make_async_copy(table.at[idxg[p]], rows[p], gsem).wait()

    def _fire_scatter(p, g3):
        pltpu.async_copy(rows[p], acc.at[scx[g3]], ssem, add=True)

    def _drain_scatter(p):
        pltpu.make_async_copy(rows[p], acc.at[pl.ds(0, GROWS * CH), :],
                              ssem).wait()

    def conv_pass(table, floc, do_deg, deg_mul):
        zv = jnp.zeros((L,), jnp.int32)
        pv = jnp.full((L,), NP - 1, jnp.int32)

        @pl.loop(0, GROWS * (CH // L))
        def _(v):
            fl = pl.ds(v * L, L)
            idxg[1][fl] = zv
            scx[2][fl] = pv
            scx[3][fl] = pv
        _stage(0, 0)
        _stage(1, 1)
        _fire_gather(table, 1)
        _fire_scatter(0, 2)

        def process_group(g, k):
            p = k & 1
            _wait_stage(p)

            @pl.loop(0, GROWS * (CH // L))
            def _(v):
                j = v >> 3
                sl = pl.ds((v & 7) * L, L)
                fl = pl.ds(v * L, L)
                s = epk[p, j, 0, sl]
                d = epk[p, j, 1, sl]
                t = epk[p, j, 2, sl]
                idxg[p][fl] = t * NP + s + voff1
                scx[k][fl] = d
                if do_deg:
                    w = plsc.load_gather(floc, [t])
                    if deg_mul:
                        w = w * plsc.load_gather(degf, [s >> 10, s & (DW - 1)])
                    plsc.addupdate_scatter(degl, [d >> 10, d & (DW - 1)], w)

            gn = g + 2
            _stage(jnp.where(gn < NGRP, gn, gn - NGRP), p)

            _drain_scatter(p)
            _fire_gather(table, p)
            _wait_gather(table, p)
            _fire_scatter(1 - p, (k + 3) & 3)

        @pl.loop(0, NGRP // 4)
        def _(i):
            for k in range(4):
                process_group(4 * i + k, k)

        _wait_gather(table, 1)
        _fire_scatter(1, (NGRP - 1) & 3)
        _drain_scatter(0)
        _drain_scatter(1)
        _wait_stage(0)
        _wait_stage(1)

    def reduce_deg_to_degf():
        pltpu.sync_copy(degl, dacc.at[idx10], add=True)
        plsc.subcore_barrier()
        pltpu.sync_copy(dacc.at[pl.ds(0, DR)], degf)
        plsc.subcore_barrier()

    for hf in range(2):
        t1h = t1a if hf == 0 else t1b
        t2h = t2a if hf == 0 else t2b
        do_deg = hf == 0

        zero_acc_slice()
        if do_deg:
            _zero_2d(degl, DR, DW)
            pltpu.sync_copy(degl.at[0], dacc.at[wid % DR])
        plsc.subcore_barrier()

        conv_pass(t1h, floc1, do_deg, deg_mul=False)
        plsc.subcore_barrier()
        if do_deg:
            reduce_deg_to_degf()
            _zero_2d(degl, DR, DW)
            pltpu.sync_copy(degl.at[0], dacc.at[wid % DR])

        fv2 = floc2[...]
        lanes = lax.iota(jnp.int32, L)
        for t in range(NUM_ETYPES):
            wt = jnp.sum(jnp.where(lanes == t, fv2, 0.0))
            wtv = jnp.full((L,), wt, jnp.float32)
            for sb_i in range(RPT // SBLK):
                rbase = base + sb_i * SBLK
                pltpu.sync_copy(acc.at[pl.ds(rbase, SBLK), :], sbuf)

                @pl.loop(0, SBLK)
                def _(r):
                    for j in range(DH // L):
                        sl = pl.ds(j * L, L)
                        sbuf[r, sl] = sbuf[r, sl] * wtv
                pltpu.sync_copy(
                    sbuf, t2h.at[pl.ds((cid * NUM_ETYPES + t) * NP + rbase, SBLK), :])

        zero_acc_slice()
        plsc.subcore_barrier()

        conv_pass(t2h, floc2, do_deg, deg_mul=True)
        plsc.subcore_barrier()
        if do_deg:
            reduce_deg_to_degf()

        if do_deg:
            @pl.loop(0, RPT // L)
            def _(jj):
                idx = base + jj * L + iotab[...]
                dv = plsc.load_gather(degf, [idx >> 10, idx & (DW - 1)])
                rec = jnp.where(dv == 0.0, 0.0,
                                1.0 / jnp.where(dv == 0.0, 1.0, dv))
                recrow[pl.ds(jj * L, L)] = rec

        for sb_i in range(RPT // SBLK):
            rbase = base + sb_i * SBLK
            pltpu.sync_copy(acc.at[pl.ds(rbase, SBLK), :], sbuf)

            @pl.loop(0, SBLK)
            def _(r):
                wsp = plsc.load_gather(
                    recrow, [jnp.full((L,), sb_i * SBLK + r, jnp.int32)])
                for j in range(DH // L):
                    sl = pl.ds(j * L, L)
                    sbuf[r, sl] = sbuf[r, sl] * wsp
            pltpu.sync_copy(sbuf, zz.at[cid, hf, pl.ds(rbase, SBLK), :])


def _sc_conv(t1a, t1b, epkv, iotav, filtv):
    mesh = plsc.VectorSubcoreMesh(core_axis_name="c", subcore_axis_name="s")
    kfn = pl.kernel(
        _sc_body,
        out_type=[
            jax.ShapeDtypeStruct((HEADS, 2, NP, DH), jnp.float32),
            jax.ShapeDtypeStruct((NV * NP, DH), jnp.float32),
            jax.ShapeDtypeStruct((NV * NP, DH), jnp.float32),
        ],
        mesh=mesh,
        scratch_types=[
            pltpu.VMEM((2, GROWS, 3, CH), jnp.int32),
            pltpu.VMEM((GROWS * CH,), jnp.int32),
            pltpu.VMEM((GROWS * CH,), jnp.int32),
            pltpu.VMEM((GROWS * CH,), jnp.int32),
            pltpu.VMEM((GROWS * CH,), jnp.int32),
            pltpu.VMEM((GROWS * CH,), jnp.int32),
            pltpu.VMEM((GROWS * CH,), jnp.int32),
            pltpu.VMEM((GROWS * CH, DH), jnp.float32),
            pltpu.VMEM((GROWS * CH, DH), jnp.float32),
            pltpu.VMEM((SBLK, DH), jnp.float32),
            pltpu.VMEM((DR, DW), jnp.float32),
            pltpu.VMEM((DR, DW), jnp.float32),
            pltpu.VMEM((L,), jnp.float32),
            pltpu.VMEM((L,), jnp.float32),
            pltpu.VMEM((L,), jnp.int32),
            pltpu.VMEM((DR,), jnp.int32),
            pltpu.VMEM((RPT,), jnp.float32),
            pltpu.VMEM_SHARED((NP, DH), jnp.float32),
            pltpu.VMEM_SHARED((NS, DW), jnp.float32),
            pltpu.SemaphoreType.DMA,
            pltpu.SemaphoreType.DMA,
            pltpu.SemaphoreType.DMA,
        ],
        compiler_params=pltpu.CompilerParams(needs_layout_passes=False,
                                             use_tc_tiling_on_sc=False),
    )
    return kfn(t1a, t1b, epkv, iotav, filtv)


_BLK_C = 1280


def _head_body(z_ref, w2_ref, b2_ref, w3t_ref, b3_ref, logits_ref, enc_ref):
    enc = b2_ref[...]
    for hd in range(HEADS):
        for hf in range(2):
            zp = jax.nn.relu(z_ref[hd, hf, :, :])
            wq = w2_ref[:, (hd * 2 + hf) * DH:(hd * 2 + hf + 1) * DH]
            enc = enc + jnp.dot(zp, wq.T, preferred_element_type=jnp.float32)
    enc_ref[...] = enc
    logits_ref[...] = (
        jnp.dot(jax.nn.relu(enc), w3t_ref[...], preferred_element_type=jnp.float32)
        + b3_ref[...])


def _head(zz, w2, b2, w3t, b3, ncls):
    nblk = NP // _BLK_C
    return pl.pallas_call(
        _head_body,
        grid=(nblk,),
        in_specs=[
            pl.BlockSpec((HEADS, 2, _BLK_C, DH), lambda i: (0, 0, i, 0)),
            pl.BlockSpec((D, 2 * D), lambda i: (0, 0)),
            pl.BlockSpec((1, D), lambda i: (0, 0)),
            pl.BlockSpec((D, ncls), lambda i: (0, 0)),
            pl.BlockSpec((1, ncls), lambda i: (0, 0)),
        ],
        out_specs=[
            pl.BlockSpec((_BLK_C, ncls), lambda i: (i, 0)),
            pl.BlockSpec((_BLK_C, D), lambda i: (i, 0)),
        ],
        out_shape=[
            jax.ShapeDtypeStruct((NP, ncls), jnp.float32),
            jax.ShapeDtypeStruct((NP, D), jnp.float32),
        ],
    )(zz, w2, b2, w3t, b3)


def kernel(feat0, feat1, edge_index, edge_type, e_feat,
           fc1_w0, fc1_b0, fc1_w1, fc1_b1,
           fc2_w, fc2_b, fc3_w, fc3_b, conv_w):
    del e_feat
    filt = jax.nn.softmax(conv_w)

    featp = jnp.zeros((NP, D), jnp.float32)
    featp = featp.at[:N0].set(feat0).at[N0:N].set(feat1)

    scales1 = jnp.broadcast_to(filt[:, 0, :].reshape(NV, 1), (NV, D))
    t1 = _fc1_scaled_tables(featp, fc1_w0, fc1_b0.reshape(1, D),
                            fc1_w1, fc1_b1.reshape(1, D), scales1)
    t1a = t1[0].reshape(NV * NP, DH)
    t1b = t1[1].reshape(NV * NP, DH)

    filtv = jnp.zeros((HEADS, 2, L), jnp.float32).at[:, :, :NUM_ETYPES].set(filt)
    src2 = jnp.zeros((E2,), jnp.int32).at[:E].set(edge_index[0]).reshape(E2 // CH, CH)
    dst2 = jnp.full((E2,), NP - 1, jnp.int32).at[:E].set(edge_index[1]).reshape(E2 // CH, CH)
    et2 = jnp.zeros((E2,), jnp.int32).at[:E].set(edge_type).reshape(E2 // CH, CH)
    epkv = jnp.stack([src2, dst2, et2], axis=1)
    iotav = jnp.arange(L, dtype=jnp.int32)

    zz, _t2a, _t2b = _sc_conv(t1a, t1b, epkv, iotav, filtv)

    ncls = fc3_w.shape[0]
    logits_p, enc_p = _head(zz, fc2_w, fc2_b.reshape(1, D), fc3_w.T,
                            fc3_b.reshape(1, ncls), ncls)
    return (logits_p[:N], enc_p[:N])


# speedup vs baseline: 1.2702x; 1.2702x over previous
import jax
import jax.numpy as jnp
from jax import lax
from jax.experimental import pallas as pl
from jax.experimental.pallas import tpu as pltpu
from jax.experimental.pallas import tpu_sc as plsc

N0 = 5000
N = 10000
E = 320000
D = 128
DH = 64
NUM_ETYPES = 4
HEADS = 2

NC = 2
NS = 16
L = 16

NP = 10240
RPT = NP // NS

E2 = 327680
EPT = E2 // NS
CH = 64
NR2 = EPT // CH
GROWS = 4
NGRP = NR2 // GROWS
SBLK = 64

DW = 1024
DR = NP // DW
NV = HEADS * NUM_ETYPES


_BLK_A = 1280


def _fc1_body(feat_ref, w0_ref, b0_ref, w1_ref, b1_ref, sc_ref, out_ref):
    i = pl.program_id(0)
    feat = feat_ref[...]
    h0 = jnp.dot(feat, w0_ref[...].T, preferred_element_type=jnp.float32) + b0_ref[...]
    h1 = jnp.dot(feat, w1_ref[...].T, preferred_element_type=jnp.float32) + b1_ref[...]
    rows = jax.lax.broadcasted_iota(jnp.int32, (_BLK_A, 1), 0) + i * _BLK_A
    h = jnp.where(rows < N0, h0, h1)
    for hf in range(2):
        hh = h[:, hf * DH:(hf + 1) * DH]
        for v in range(NV):
            out_ref[hf, v, :, :] = hh * sc_ref[v:v + 1, :DH]


def _fc1_scaled_tables(featp, w0, b0, w1, b1, scales):
    nblk = NP // _BLK_A
    return pl.pallas_call(
        _fc1_body,
        grid=(nblk,),
        in_specs=[
            pl.BlockSpec((_BLK_A, D), lambda i: (i, 0)),
            pl.BlockSpec((D, D), lambda i: (0, 0)),
            pl.BlockSpec((1, D), lambda i: (0, 0)),
            pl.BlockSpec((D, D), lambda i: (0, 0)),
            pl.BlockSpec((1, D), lambda i: (0, 0)),
            pl.BlockSpec((NV, D), lambda i: (0, 0)),
        ],
        out_specs=pl.BlockSpec((2, NV, _BLK_A, DH), lambda i: (0, 0, i, 0)),
        out_shape=jax.ShapeDtypeStruct((2, NV, NP, DH), jnp.float32),
    )(featp, w0, b0, w1, b1, scales)


def _zero_2d(ref, nrows, ncols):
    z = jnp.zeros((L,), jnp.float32)
    for r in range(nrows):
        @pl.loop(0, ncols // L)
        def _(j):
            ref[r, pl.ds(j * L, L)] = z


def _sc_body(t1a, t1b, epkv, iotav, filtv, zz, t2a, t2b,
             epk, idx_a, idx_b, sc_a, sc_b, rows_a, rows_b, sbuf,
             degl, degf, floc1, floc2, iotab, idx10, recrow,
             acc, dacc, stsem, gsem, ssem):
    idxg = [idx_a, idx_b]
    scx = [sc_a, sc_b]
    rows = [rows_a, rows_b]
    cid = lax.axis_index("c")
    wid = lax.axis_index("s")
    base = wid * RPT

    pltpu.sync_copy(filtv.at[cid, 0], floc1)
    pltpu.sync_copy(filtv.at[cid, 1], floc2)
    iotab[...] = lax.iota(jnp.int32, L)
    pltpu.sync_copy(iotav.at[pl.ds(0, DR)], idx10)

    voff1 = (cid * NUM_ETYPES) * NP

    def zero_acc_slice():
        _zero_2d(sbuf, SBLK, DH)
        for k in range(RPT // SBLK):
            pltpu.sync_copy(sbuf, acc.at[pl.ds(base + k * SBLK, SBLK), :])

    def _stage(g, p):
        rb = wid * NR2 + g * GROWS
        pltpu.async_copy(epkv.at[pl.ds(rb, GROWS)], epk.at[p], stsem)

    def _wait_stage(p):
        sl = pl.ds(wid * NR2, GROWS)
        pltpu.make_async_copy(epkv.at[sl], epk.at[p], stsem).wait()

    def _fire_gathers(table, p):
        for j in range(GROWS):
            pltpu.async_copy(table.at[idxg[p].at[pl.ds(j * CH, CH)]],
                             rows[p].at[pl.ds(j * CH, CH), :], gsem)

    def _drain_gathers(table, p):
        pltpu.make_async_copy(table.at[pl.ds(0, GROWS * CH), :], rows[p],
                              gsem).wait()

    def _fire_scatters(p):
        for j in range(GROWS):
            pltpu.async_copy(rows[p].at[pl.ds(j * CH, CH), :],
                             acc.at[scx[p].at[j]], ssem, add=True)

    def _drain_scatters(p):
        pltpu.make_async_copy(rows[p], acc.at[pl.ds(0, GROWS * CH), :],
                              ssem).wait()

    def conv_pass(table, floc, do_deg, deg_mul):
        pv = jnp.full((L,), NP - 1, jnp.int32)

        @pl.loop(0, GROWS * (CH // L))
        def _(v):
            j = v >> 2
            scx[1][j, pl.ds((v & 3) * L, L)] = pv
        _stage(0, 0)
        _stage(1, 1)
        _fire_scatters(1)

        def process_group(g, p):
            _wait_stage(p)

            @pl.loop(0, GROWS * (CH // L))
            def _(v):
                j = v >> 2
                sl = pl.ds((v & 3) * L, L)
                fl = pl.ds(v * L, L)
                s = epk[p, j, 0, sl]
                d = epk[p, j, 1, sl]
                t = epk[p, j, 2, sl]
                idxg[p][fl] = t * NP + s + voff1
                scx[p][j, sl] = d
                if do_deg:
                    w = plsc.load_gather(floc, [t])
                    if deg_mul:
                        w = w * plsc.load_gather(degf, [s >> 10, s & (DW - 1)])
                    plsc.addupdate_scatter(degl, [d >> 10, d & (DW - 1)], w)

            gn = g + 2
            _stage(jnp.where(gn < NGRP, gn, gn - NGRP), p)

            _fire_gathers(table, p)
            _drain_gathers(table, p)
            _drain_scatters(1 - p)
            _fire_scatters(p)

        @pl.loop(0, NGRP // 2)
        def _(i):
            process_group(2 * i, 0)
            process_group(2 * i + 1, 1)

        _drain_scatters(1)
        _wait_stage(0)
        _wait_stage(1)

    def reduce_deg_to_degf():
        pltpu.sync_copy(degl, dacc.at[idx10], add=True)
        plsc.subcore_barrier()
        pltpu.sync_copy(dacc.at[pl.ds(0, DR)], degf)
        plsc.subcore_barrier()

    for hf in range(2):
        t1h = t1a if hf == 0 else t1b
        t2h = t2a if hf == 0 else t2b
        do_deg = hf == 0

        zero_acc_slice()
        if do_deg:
            _zero_2d(degl, DR, DW)
            pltpu.sync_copy(degl.at[0], dacc.at[wid % DR])
        plsc.subcore_barrier()

        conv_pass(t1h, floc1, do_deg, deg_mul=False)
        plsc.subcore_barrier()
        if do_deg:
            reduce_deg_to_degf()
            _zero_2d(degl, DR, DW)
            pltpu.sync_copy(degl.at[0], dacc.at[wid % DR])

        fv2 = floc2[...]
        lanes = lax.iota(jnp.int32, L)
        for t in range(NUM_ETYPES):
            wt = jnp.sum(jnp.where(lanes == t, fv2, 0.0))
            wtv = jnp.full((L,), wt, jnp.float32)
            for sb_i in range(RPT // SBLK):
                rbase = base + sb_i * SBLK
                pltpu.sync_copy(acc.at[pl.ds(rbase, SBLK), :], sbuf)

                @pl.loop(0, SBLK)
                def _(r):
                    for j in range(DH // L):
                        sl = pl.ds(j * L, L)
                        sbuf[r, sl] = sbuf[r, sl] * wtv
                pltpu.sync_copy(
                    sbuf, t2h.at[pl.ds((cid * NUM_ETYPES + t) * NP + rbase, SBLK), :])

        zero_acc_slice()
        plsc.subcore_barrier()

        conv_pass(t2h, floc2, do_deg, deg_mul=True)
        plsc.subcore_barrier()
        if do_deg:
            reduce_deg_to_degf()

        if do_deg:
            @pl.loop(0, RPT // L)
            def _(jj):
                idx = base + jj * L + iotab[...]
                dv = plsc.load_gather(degf, [idx >> 10, idx & (DW - 1)])
                rec = jnp.where(dv == 0.0, 0.0,
                                1.0 / jnp.where(dv == 0.0, 1.0, dv))
                recrow[pl.ds(jj * L, L)] = rec

        for sb_i in range(RPT // SBLK):
            rbase = base + sb_i * SBLK
            pltpu.sync_copy(acc.at[pl.ds(rbase, SBLK), :], sbuf)

            @pl.loop(0, SBLK)
            def _(r):
                wsp = plsc.load_gather(
                    recrow, [jnp.full((L,), sb_i * SBLK + r, jnp.int32)])
                for j in range(DH // L):
                    sl = pl.ds(j * L, L)
                    sbuf[r, sl] = sbuf[r, sl] * wsp
            pltpu.sync_copy(sbuf, zz.at[cid, hf, pl.ds(rbase, SBLK), :])


def _sc_conv(t1a, t1b, epkv, iotav, filtv):
    mesh = plsc.VectorSubcoreMesh(core_axis_name="c", subcore_axis_name="s")
    kfn = pl.kernel(
        _sc_body,
        out_type=[
            jax.ShapeDtypeStruct((HEADS, 2, NP, DH), jnp.float32),
            jax.ShapeDtypeStruct((NV * NP, DH), jnp.float32),
            jax.ShapeDtypeStruct((NV * NP, DH), jnp.float32),
        ],
        mesh=mesh,
        scratch_types=[
            pltpu.VMEM((2, GROWS, 3, CH), jnp.int32),
            pltpu.VMEM((GROWS * CH,), jnp.int32),
            pltpu.VMEM((GROWS * CH,), jnp.int32),
            pltpu.VMEM((GROWS, CH), jnp.int32),
            pltpu.VMEM((GROWS, CH), jnp.int32),
            pltpu.VMEM((GROWS * CH, DH), jnp.float32),
            pltpu.VMEM((GROWS * CH, DH), jnp.float32),
            pltpu.VMEM((SBLK, DH), jnp.float32),
            pltpu.VMEM((DR, DW), jnp.float32),
            pltpu.VMEM((DR, DW), jnp.float32),
            pltpu.VMEM((L,), jnp.float32),
            pltpu.VMEM((L,), jnp.float32),
            pltpu.VMEM((L,), jnp.int32),
            pltpu.VMEM((DR,), jnp.int32),
            pltpu.VMEM((RPT,), jnp.float32),
            pltpu.VMEM_SHARED((NP, DH), jnp.float32),
            pltpu.VMEM_SHARED((NS, DW), jnp.float32),
            pltpu.SemaphoreType.DMA,
            pltpu.SemaphoreType.DMA,
            pltpu.SemaphoreType.DMA,
        ],
        compiler_params=pltpu.CompilerParams(needs_layout_passes=False,
                                             use_tc_tiling_on_sc=False),
    )
    return kfn(t1a, t1b, epkv, iotav, filtv)


_BLK_C = 1280


def _head_body(z_ref, w2_ref, b2_ref, w3t_ref, b3_ref, logits_ref, enc_ref):
    enc = b2_ref[...]
    for hd in range(HEADS):
        for hf in range(2):
            zp = jax.nn.relu(z_ref[hd, hf, :, :])
            wq = w2_ref[:, (hd * 2 + hf) * DH:(hd * 2 + hf + 1) * DH]
            enc = enc + jnp.dot(zp, wq.T, preferred_element_type=jnp.float32)
    enc_ref[...] = enc
    logits_ref[...] = (
        jnp.dot(jax.nn.relu(enc), w3t_ref[...], preferred_element_type=jnp.float32)
        + b3_ref[...])


def _head(zz, w2, b2, w3t, b3, ncls):
    nblk = NP // _BLK_C
    return pl.pallas_call(
        _head_body,
        grid=(nblk,),
        in_specs=[
            pl.BlockSpec((HEADS, 2, _BLK_C, DH), lambda i: (0, 0, i, 0)),
            pl.BlockSpec((D, 2 * D), lambda i: (0, 0)),
            pl.BlockSpec((1, D), lambda i: (0, 0)),
            pl.BlockSpec((D, ncls), lambda i: (0, 0)),
            pl.BlockSpec((1, ncls), lambda i: (0, 0)),
        ],
        out_specs=[
            pl.BlockSpec((_BLK_C, ncls), lambda i: (i, 0)),
            pl.BlockSpec((_BLK_C, D), lambda i: (i, 0)),
        ],
        out_shape=[
            jax.ShapeDtypeStruct((NP, ncls), jnp.float32),
            jax.ShapeDtypeStruct((NP, D), jnp.float32),
        ],
    )(zz, w2, b2, w3t, b3)


def kernel(feat0, feat1, edge_index, edge_type, e_feat,
           fc1_w0, fc1_b0, fc1_w1, fc1_b1,
           fc2_w, fc2_b, fc3_w, fc3_b, conv_w):
    del e_feat
    filt = jax.nn.softmax(conv_w)

    featp = jnp.zeros((NP, D), jnp.float32)
    featp = featp.at[:N0].set(feat0).at[N0:N].set(feat1)

    scales1 = jnp.broadcast_to(filt[:, 0, :].reshape(NV, 1), (NV, D))
    t1 = _fc1_scaled_tables(featp, fc1_w0, fc1_b0.reshape(1, D),
                            fc1_w1, fc1_b1.reshape(1, D), scales1)
    t1a = t1[0].reshape(NV * NP, DH)
    t1b = t1[1].reshape(NV * NP, DH)

    filtv = jnp.zeros((HEADS, 2, L), jnp.float32).at[:, :, :NUM_ETYPES].set(filt)
    src2 = jnp.zeros((E2,), jnp.int32).at[:E].set(edge_index[0]).reshape(E2 // CH, CH)
    dst2 = jnp.full((E2,), NP - 1, jnp.int32).at[:E].set(edge_index[1]).reshape(E2 // CH, CH)
    et2 = jnp.zeros((E2,), jnp.int32).at[:E].set(edge_type).reshape(E2 // CH, CH)
    epkv = jnp.stack([src2, dst2, et2], axis=1)
    iotav = jnp.arange(L, dtype=jnp.int32)

    zz, _t2a, _t2b = _sc_conv(t1a, t1b, epkv, iotav, filtv)

    ncls = fc3_w.shape[0]
    logits_p, enc_p = _head(zz, fc2_w, fc2_b.reshape(1, D), fc3_w.T,
                            fc3_b.reshape(1, ncls), ncls)
    return (logits_p[:N], enc_p[:N])


# speedup vs baseline: 1.2826x; 1.0097x over previous
import jax
import jax.numpy as jnp
from jax import lax
from jax.experimental import pallas as pl
from jax.experimental.pallas import tpu as pltpu
from jax.experimental.pallas import tpu_sc as plsc

N0 = 5000
N = 10000
E = 320000
D = 128
DH = 64
NUM_ETYPES = 4
HEADS = 2

NC = 2
NS = 16
L = 16

NP = 10240
RPT = NP // NS

E2 = 327680
EPT = E2 // NS
CH = 64
NR2 = EPT // CH
GROWS = 5
NGRP = NR2 // GROWS
SBLK = 64

DW = 1024
DR = NP // DW
NV = HEADS * NUM_ETYPES


_BLK_A = 1280


def _fc1_body(feat_ref, w0_ref, b0_ref, w1_ref, b1_ref, sc_ref, out_ref):
    i = pl.program_id(0)
    feat = feat_ref[...]
    h0 = jnp.dot(feat, w0_ref[...].T, preferred_element_type=jnp.float32) + b0_ref[...]
    h1 = jnp.dot(feat, w1_ref[...].T, preferred_element_type=jnp.float32) + b1_ref[...]
    rows = jax.lax.broadcasted_iota(jnp.int32, (_BLK_A, 1), 0) + i * _BLK_A
    h = jnp.where(rows < N0, h0, h1)
    for hf in range(2):
        hh = h[:, hf * DH:(hf + 1) * DH]
        for v in range(NV):
            out_ref[hf, v, :, :] = hh * sc_ref[v:v + 1, :DH]


def _fc1_scaled_tables(featp, w0, b0, w1, b1, scales):
    nblk = NP // _BLK_A
    return pl.pallas_call(
        _fc1_body,
        grid=(nblk,),
        in_specs=[
            pl.BlockSpec((_BLK_A, D), lambda i: (i, 0)),
            pl.BlockSpec((D, D), lambda i: (0, 0)),
            pl.BlockSpec((1, D), lambda i: (0, 0)),
            pl.BlockSpec((D, D), lambda i: (0, 0)),
            pl.BlockSpec((1, D), lambda i: (0, 0)),
            pl.BlockSpec((NV, D), lambda i: (0, 0)),
        ],
        out_specs=pl.BlockSpec((2, NV, _BLK_A, DH), lambda i: (0, 0, i, 0)),
        out_shape=jax.ShapeDtypeStruct((2, NV, NP, DH), jnp.float32),
    )(featp, w0, b0, w1, b1, scales)


def _zero_2d(ref, nrows, ncols):
    z = jnp.zeros((L,), jnp.float32)
    for r in range(nrows):
        @pl.loop(0, ncols // L)
        def _(j):
            ref[r, pl.ds(j * L, L)] = z


def _sc_body(t1a, t1b, epkv, iotav, filtv, zz, t2a, t2b,
             epk, idx_a, idx_b, sc_a, sc_b, rows_a, rows_b, sbuf,
             degl, degf, floc1, floc2, iotab, idx10, recrow,
             acc, dacc, stsem, gsem, ssem):
    idxg = [idx_a, idx_b]
    scx = [sc_a, sc_b]
    rows = [rows_a, rows_b]
    cid = lax.axis_index("c")
    wid = lax.axis_index("s")
    base = wid * RPT

    pltpu.sync_copy(filtv.at[cid, 0], floc1)
    pltpu.sync_copy(filtv.at[cid, 1], floc2)
    iotab[...] = lax.iota(jnp.int32, L)
    pltpu.sync_copy(iotav.at[pl.ds(0, DR)], idx10)

    voff1 = (cid * NUM_ETYPES) * NP

    def zero_acc_slice():
        _zero_2d(sbuf, SBLK, DH)
        for k in range(RPT // SBLK):
            pltpu.sync_copy(sbuf, acc.at[pl.ds(base + k * SBLK, SBLK), :])

    def _stage(g, p):
        rb = wid * NR2 + g * GROWS
        pltpu.async_copy(epkv.at[pl.ds(rb, GROWS)], epk.at[p], stsem)

    def _wait_stage(p):
        sl = pl.ds(wid * NR2, GROWS)
        pltpu.make_async_copy(epkv.at[sl], epk.at[p], stsem).wait()

    def _fire_gathers(table, p):
        for j in range(GROWS):
            pltpu.async_copy(table.at[idxg[p].at[pl.ds(j * CH, CH)]],
                             rows[p].at[pl.ds(j * CH, CH), :], gsem)

    def _drain_gathers(table, p):
        pltpu.make_async_copy(table.at[pl.ds(0, GROWS * CH), :], rows[p],
                              gsem).wait()

    def _fire_scatters(p):
        for j in range(GROWS):
            pltpu.async_copy(rows[p].at[pl.ds(j * CH, CH), :],
                             acc.at[scx[p].at[j]], ssem, add=True)

    def _drain_scatters(p):
        pltpu.make_async_copy(rows[p], acc.at[pl.ds(0, GROWS * CH), :],
                              ssem).wait()

    def conv_pass(table, floc, do_deg, deg_mul):
        pv = jnp.full((L,), NP - 1, jnp.int32)

        @pl.loop(0, GROWS * (CH // L))
        def _(v):
            j = v >> 2
            scx[1][j, pl.ds((v & 3) * L, L)] = pv
        _stage(0, 0)
        _stage(1, 1)
        _fire_scatters(1)

        def process_group(g, p):
            _wait_stage(p)

            @pl.loop(0, GROWS * (CH // L))
            def _(v):
                j = v >> 2
                sl = pl.ds((v & 3) * L, L)
                fl = pl.ds(v * L, L)
                s = epk[p, j, 0, sl]
                d = epk[p, j, 1, sl]
                t = epk[p, j, 2, sl]
                idxg[p][fl] = t * NP + s + voff1
                scx[p][j, sl] = d
                if do_deg:
                    w = plsc.load_gather(floc, [t])
                    if deg_mul:
                        w = w * plsc.load_gather(degf, [s >> 10, s & (DW - 1)])
                    plsc.addupdate_scatter(degl, [d >> 10, d & (DW - 1)], w)

            gn = g + 2
            _stage(jnp.where(gn < NGRP, gn, gn - NGRP), p)

            _fire_gathers(table, p)
            _drain_gathers(table, p)
            _drain_scatters(1 - p)
            _fire_scatters(p)

        @pl.loop(0, NGRP // 2)
        def _(i):
            process_group(2 * i, 0)
            process_group(2 * i + 1, 1)

        _drain_scatters(1)
        _wait_stage(0)
        _wait_stage(1)

    def reduce_deg_to_degf():
        pltpu.sync_copy(degl, dacc.at[idx10], add=True)
        plsc.subcore_barrier()
        pltpu.sync_copy(dacc.at[pl.ds(0, DR)], degf)
        plsc.subcore_barrier()

    for hf in range(2):
        t1h = t1a if hf == 0 else t1b
        t2h = t2a if hf == 0 else t2b
        do_deg = hf == 0

        zero_acc_slice()
        if do_deg:
            _zero_2d(degl, DR, DW)
            pltpu.sync_copy(degl.at[0], dacc.at[wid % DR])
        plsc.subcore_barrier()

        conv_pass(t1h, floc1, do_deg, deg_mul=False)
        plsc.subcore_barrier()
        if do_deg:
            reduce_deg_to_degf()
            _zero_2d(degl, DR, DW)
            pltpu.sync_copy(degl.at[0], dacc.at[wid % DR])

        fv2 = floc2[...]
        lanes = lax.iota(jnp.int32, L)
        for t in range(NUM_ETYPES):
            wt = jnp.sum(jnp.where(lanes == t, fv2, 0.0))
            wtv = jnp.full((L,), wt, jnp.float32)
            for sb_i in range(RPT // SBLK):
                rbase = base + sb_i * SBLK
                pltpu.sync_copy(acc.at[pl.ds(rbase, SBLK), :], sbuf)

                @pl.loop(0, SBLK)
                def _(r):
                    for j in range(DH // L):
                        sl = pl.ds(j * L, L)
                        sbuf[r, sl] = sbuf[r, sl] * wtv
                pltpu.sync_copy(
                    sbuf, t2h.at[pl.ds((cid * NUM_ETYPES + t) * NP + rbase, SBLK), :])

        zero_acc_slice()
        plsc.subcore_barrier()

        conv_pass(t2h, floc2, do_deg, deg_mul=True)
        plsc.subcore_barrier()
        if do_deg:
            reduce_deg_to_degf()

        if do_deg:
            @pl.loop(0, RPT // L)
            def _(jj):
                idx = base + jj * L + iotab[...]
                dv = plsc.load_gather(degf, [idx >> 10, idx & (DW - 1)])
                rec = jnp.where(dv == 0.0, 0.0,
                                1.0 / jnp.where(dv == 0.0, 1.0, dv))
                recrow[pl.ds(jj * L, L)] = rec

        for sb_i in range(RPT // SBLK):
            rbase = base + sb_i * SBLK
            pltpu.sync_copy(acc.at[pl.ds(rbase, SBLK), :], sbuf)

            @pl.loop(0, SBLK)
            def _(r):
                wsp = plsc.load_gather(
                    recrow, [jnp.full((L,), sb_i * SBLK + r, jnp.int32)])
                for j in range(DH // L):
                    sl = pl.ds(j * L, L)
                    sbuf[r, sl] = sbuf[r, sl] * wsp
            pltpu.sync_copy(sbuf, zz.at[cid, hf, pl.ds(rbase, SBLK), :])


def _sc_conv(t1a, t1b, epkv, iotav, filtv):
    mesh = plsc.VectorSubcoreMesh(core_axis_name="c", subcore_axis_name="s")
    kfn = pl.kernel(
        _sc_body,
        out_type=[
            jax.ShapeDtypeStruct((HEADS, 2, NP, DH), jnp.float32),
            jax.ShapeDtypeStruct((NV * NP, DH), jnp.float32),
            jax.ShapeDtypeStruct((NV * NP, DH), jnp.float32),
        ],
        mesh=mesh,
        scratch_types=[
            pltpu.VMEM((2, GROWS, 3, CH), jnp.int32),
            pltpu.VMEM((GROWS * CH,), jnp.int32),
            pltpu.VMEM((GROWS * CH,), jnp.int32),
            pltpu.VMEM((GROWS, CH), jnp.int32),
            pltpu.VMEM((GROWS, CH), jnp.int32),
            pltpu.VMEM((GROWS * CH, DH), jnp.float32),
            pltpu.VMEM((GROWS * CH, DH), jnp.float32),
            pltpu.VMEM((SBLK, DH), jnp.float32),
            pltpu.VMEM((DR, DW), jnp.float32),
            pltpu.VMEM((DR, DW), jnp.float32),
            pltpu.VMEM((L,), jnp.float32),
            pltpu.VMEM((L,), jnp.float32),
            pltpu.VMEM((L,), jnp.int32),
            pltpu.VMEM((DR,), jnp.int32),
            pltpu.VMEM((RPT,), jnp.float32),
            pltpu.VMEM_SHARED((NP, DH), jnp.float32),
            pltpu.VMEM_SHARED((NS, DW), jnp.float32),
            pltpu.SemaphoreType.DMA,
            pltpu.SemaphoreType.DMA,
            pltpu.SemaphoreType.DMA,
        ],
        compiler_params=pltpu.CompilerParams(needs_layout_passes=False,
                                             use_tc_tiling_on_sc=False),
    )
    return kfn(t1a, t1b, epkv, iotav, filtv)


_BLK_C = 1280


def _head_body(z_ref, w2_ref, b2_ref, w3t_ref, b3_ref, logits_ref, enc_ref):
    enc = b2_ref[...]
    for hd in range(HEADS):
        for hf in range(2):
            zp = jax.nn.relu(z_ref[hd, hf, :, :])
            wq = w2_ref[:, (hd * 2 + hf) * DH:(hd * 2 + hf + 1) * DH]
            enc = enc + jnp.dot(zp, wq.T, preferred_element_type=jnp.float32)
    enc_ref[...] = enc
    logits_ref[...] = (
        jnp.dot(jax.nn.relu(enc), w3t_ref[...], preferred_element_type=jnp.float32)
        + b3_ref[...])


def _head(zz, w2, b2, w3t, b3, ncls):
    nblk = NP // _BLK_C
    return pl.pallas_call(
        _head_body,
        grid=(nblk,),
        in_specs=[
            pl.BlockSpec((HEADS, 2, _BLK_C, DH), lambda i: (0, 0, i, 0)),
            pl.BlockSpec((D, 2 * D), lambda i: (0, 0)),
            pl.BlockSpec((1, D), lambda i: (0, 0)),
            pl.BlockSpec((D, ncls), lambda i: (0, 0)),
            pl.BlockSpec((1, ncls), lambda i: (0, 0)),
        ],
        out_specs=[
            pl.BlockSpec((_BLK_C, ncls), lambda i: (i, 0)),
            pl.BlockSpec((_BLK_C, D), lambda i: (i, 0)),
        ],
        out_shape=[
            jax.ShapeDtypeStruct((NP, ncls), jnp.float32),
            jax.ShapeDtypeStruct((NP, D), jnp.float32),
        ],
    )(zz, w2, b2, w3t, b3)


def kernel(feat0, feat1, edge_index, edge_type, e_feat,
           fc1_w0, fc1_b0, fc1_w1, fc1_b1,
           fc2_w, fc2_b, fc3_w, fc3_b, conv_w):
    del e_feat
    filt = jax.nn.softmax(conv_w)

    featp = jnp.zeros((NP, D), jnp.float32)
    featp = featp.at[:N0].set(feat0).at[N0:N].set(feat1)

    scales1 = jnp.broadcast_to(filt[:, 0, :].reshape(NV, 1), (NV, D))
    t1 = _fc1_scaled_tables(featp, fc1_w0, fc1_b0.reshape(1, D),
                            fc1_w1, fc1_b1.reshape(1, D), scales1)
    t1a = t1[0].reshape(NV * NP, DH)
    t1b = t1[1].reshape(NV * NP, DH)

    filtv = jnp.zeros((HEADS, 2, L), jnp.float32).at[:, :, :NUM_ETYPES].set(filt)
    src2 = jnp.zeros((E2,), jnp.int32).at[:E].set(edge_index[0]).reshape(E2 // CH, CH)
    dst2 = jnp.full((E2,), NP - 1, jnp.int32).at[:E].set(edge_index[1]).reshape(E2 // CH, CH)
    et2 = jnp.zeros((E2,), jnp.int32).at[:E].set(edge_type).reshape(E2 // CH, CH)
    epkv = jnp.stack([src2, dst2, et2], axis=1)
    iotav = jnp.arange(L, dtype=jnp.int32)

    zz, _t2a, _t2b = _sc_conv(t1a, t1b, epkv, iotav, filtv)

    ncls = fc3_w.shape[0]
    logits_p, enc_p = _head(zz, fc2_w, fc2_b.reshape(1, D), fc3_w.T,
                            fc3_b.reshape(1, ncls), ncls)
    return (logits_p[:N], enc_p[:N])
